# Initial kernel scaffold; baseline (speedup 1.0000x reference)
import jax
import jax.numpy as jnp
from jax.experimental import pallas as pl


def kernel(features, edge_index, W1, b1, W2, b2, W3, b3):
    raise NotImplementedError("write your pallas kernel here")


# speedup vs baseline: 4.3851x; 4.3851x over previous
import functools

import jax
import jax.numpy as jnp
from jax import lax
from jax.experimental import pallas as pl
from jax.experimental.pallas import tpu as pltpu
from jax.experimental.pallas import tpu_sc as plsc

N_NODES = 10000
N_EDGES = 320000
D = 128

NC = 2
NS = 16
NW = NC * NS

N_PAD = 10240
ROWS_PER_TILE = N_PAD // NS

E_PER_TILE = N_EDGES // NW
CHUNK = 80
N_CHUNKS = E_PER_TILE // CHUNK

_MESH = plsc.VectorSubcoreMesh(core_axis_name="c", subcore_axis_name="s")

_ZERO16 = None


def _zero_rows_buf(rows):
    z = jnp.zeros((16,), jnp.float32)

    def body(i, carry):
        for j in range(D // 16):
            rows[i, pl.ds(j * 16, 16)] = z
        return carry

    lax.fori_loop(0, CHUNK, body, 0)


def _degree_body(src_hbm, dst_hbm, out_hbm, idx_v, ones_v, stage_v,
                 acc_src, acc_dst, sem):
    c = lax.axis_index("c")
    s = lax.axis_index("s")
    wid = c * NS + s

    z = jnp.zeros((16,), jnp.float32)
    for j in range(CHUNK // 16):
        ones_v[pl.ds(j * 16, 16)] = jnp.ones((16,), jnp.float32)
    for j in range(ROWS_PER_TILE // 16):
        stage_v[pl.ds(j * 16, 16)] = z

    r0 = s * ROWS_PER_TILE
    pltpu.sync_copy(stage_v, acc_src.at[pl.ds(r0, ROWS_PER_TILE)])
    pltpu.sync_copy(stage_v, acc_dst.at[pl.ds(r0, ROWS_PER_TILE)])
    plsc.subcore_barrier()

    base = wid * E_PER_TILE

    def body(i, carry):
        off = base + i * CHUNK
        pltpu.sync_copy(src_hbm.at[pl.ds(off, CHUNK)], idx_v)
        pltpu.sync_copy(ones_v, acc_src.at[idx_v], add=True)
        pltpu.sync_copy(dst_hbm.at[pl.ds(off, CHUNK)], idx_v)
        pltpu.sync_copy(ones_v, acc_dst.at[idx_v], add=True)
        return carry

    lax.fori_loop(0, N_CHUNKS, body, 0)
    plsc.subcore_barrier()

    pltpu.sync_copy(acc_src.at[pl.ds(r0, ROWS_PER_TILE)], stage_v)
    pltpu.sync_copy(stage_v, out_hbm.at[c, 0, pl.ds(r0, ROWS_PER_TILE)])
    pltpu.sync_copy(acc_dst.at[pl.ds(r0, ROWS_PER_TILE)], stage_v)
    pltpu.sync_copy(stage_v, out_hbm.at[c, 1, pl.ds(r0, ROWS_PER_TILE)])


_degree_call = functools.partial(
    pl.kernel,
    mesh=_MESH,
    out_type=jax.ShapeDtypeStruct((NC, 2, N_PAD), jnp.float32),
    scratch_types=[
        pltpu.VMEM((CHUNK,), jnp.int32),
        pltpu.VMEM((CHUNK,), jnp.float32),
        pltpu.VMEM((ROWS_PER_TILE,), jnp.float32),
        pltpu.VMEM_SHARED((N_PAD,), jnp.float32),
        pltpu.VMEM_SHARED((N_PAD,), jnp.float32),
        pltpu.SemaphoreType.DMA,
    ],
)(_degree_body)


def _agg_body(y_hbm, src_hbm, dst_hbm, out_hbm, sidx, didx, rows, stage,
              acc, sem):
    c = lax.axis_index("c")
    s = lax.axis_index("s")
    wid = c * NS + s

    _zero_rows_buf(rows)
    r0 = s * ROWS_PER_TILE
    for j in range(ROWS_PER_TILE // CHUNK):
        pltpu.sync_copy(rows, acc.at[pl.ds(r0 + j * CHUNK, CHUNK)])
    plsc.subcore_barrier()

    base = wid * E_PER_TILE

    def body(i, carry):
        off = base + i * CHUNK
        pltpu.sync_copy(src_hbm.at[pl.ds(off, CHUNK)], sidx)
        pltpu.sync_copy(dst_hbm.at[pl.ds(off, CHUNK)], didx)
        pltpu.async_copy(y_hbm.at[sidx], rows, sem).wait()
        pltpu.sync_copy(rows, acc.at[didx], add=True)
        return carry

    lax.fori_loop(0, N_CHUNKS, body, 0)
    plsc.subcore_barrier()

    for j in range(ROWS_PER_TILE // CHUNK):
        pltpu.sync_copy(acc.at[pl.ds(r0 + j * CHUNK, CHUNK)], stage)
        pltpu.sync_copy(stage, out_hbm.at[c, pl.ds(r0 + j * CHUNK, CHUNK)])


_agg_call = functools.partial(
    pl.kernel,
    mesh=_MESH,
    out_type=jax.ShapeDtypeStruct((NC, N_PAD, D), jnp.float32),
    scratch_types=[
        pltpu.VMEM((CHUNK,), jnp.int32),
        pltpu.VMEM((CHUNK,), jnp.int32),
        pltpu.VMEM((CHUNK, D), jnp.float32),
        pltpu.VMEM((CHUNK, D), jnp.float32),
        pltpu.VMEM_SHARED((N_PAD, D), jnp.float32),
        pltpu.SemaphoreType.DMA,
    ],
)(_agg_body)


def _t1_body(x_ref, w_ref, dc_ref, y_ref, so_ref, si_ref):
    cs = dc_ref[0, 0] + dc_ref[1, 0]
    cd = dc_ref[0, 1] + dc_ref[1, 1]
    so = lax.rsqrt(jnp.maximum(cs, 1.0))
    si = lax.rsqrt(jnp.maximum(cd, 1.0))
    y = jnp.dot(x_ref[...], w_ref[...], preferred_element_type=jnp.float32)
    y_ref[...] = y * so
    so_ref[...] = so
    si_ref[...] = si


def _t1_call(x, w, dcnt):
    return pl.pallas_call(
        _t1_body,
        out_shape=(
            jax.ShapeDtypeStruct((N_PAD, D), jnp.float32),
            jax.ShapeDtypeStruct((N_PAD, 1), jnp.float32),
            jax.ShapeDtypeStruct((N_PAD, 1), jnp.float32),
        ),
    )(x, w, dcnt)


def _tmid_body(p_ref, si_ref, so_ref, b_ref, w_ref, y_ref):
    h = (p_ref[0] + p_ref[1]) * si_ref[...] + b_ref[...]
    h = jnp.maximum(h, 0.0)
    y = jnp.dot(h, w_ref[...], preferred_element_type=jnp.float32)
    y_ref[...] = y * so_ref[...]


def _tmid_call(p, si, so, b, w):
    return pl.pallas_call(
        _tmid_body,
        out_shape=jax.ShapeDtypeStruct((N_PAD, D), jnp.float32),
    )(p, si, so, b, w)


def _t4_body(p_ref, si_ref, b_ref, o_ref):
    o_ref[...] = (p_ref[0] + p_ref[1]) * si_ref[...] + b_ref[...]


def _t4_call(p, si, b):
    return pl.pallas_call(
        _t4_body,
        out_shape=jax.ShapeDtypeStruct((N_PAD, D), jnp.float32),
    )(p, si, b)


def kernel(features, edge_index, W1, b1, W2, b2, W3, b3):
    src = edge_index[0].astype(jnp.int32)
    dst = edge_index[1].astype(jnp.int32)
    x = jnp.pad(features, ((0, N_PAD - N_NODES), (0, 0)))

    dcnt = _degree_call(src, dst)
    dcnt = dcnt.reshape(NC, 2, N_PAD, 1)

    y1, so, si = _t1_call(x, W1, dcnt)
    p1 = _agg_call(y1, src, dst)
    y2 = _tmid_call(p1, si, so, b1.reshape(1, D), W2)
    p2 = _agg_call(y2, src, dst)
    y3 = _tmid_call(p2, si, so, b2.reshape(1, D), W3)
    p3 = _agg_call(y3, src, dst)
    out = _t4_call(p3, si, b3.reshape(1, D))
    return out[:N_NODES]


# speedup vs baseline: 9.1882x; 2.0953x over previous
import functools

import jax
import jax.numpy as jnp
from jax import lax
from jax.experimental import pallas as pl
from jax.experimental.pallas import tpu as pltpu
from jax.experimental.pallas import tpu_sc as plsc

N_NODES = 10000
N_EDGES = 320000
D = 128

NC = 2
NS = 16
NW = NC * NS

N_PAD = 10240
ROWS_PER_TILE = N_PAD // NS

E_PER_TILE = N_EDGES // NW
CHUNK = 80
N_CHUNKS = E_PER_TILE // CHUNK
N_PAIRS = (N_CHUNKS - 1) // 2

DEG_BLOCK = 8
ROWS_TOTAL = N_EDGES // CHUNK
ROWS_TOTAL_PAD = 4096
DEG_ROWS_PER_TILE = ROWS_TOTAL_PAD // NW
DEG_NBLOCKS = DEG_ROWS_PER_TILE // DEG_BLOCK

_MESH = plsc.VectorSubcoreMesh(core_axis_name="c", subcore_axis_name="s")


def _degree_body(src_hbm, dst_hbm, out_hbm, sbidx, dbidx, ones_v, stage_v,
                 acc_src, acc_dst, semi, sems):
    c = lax.axis_index("c")
    s = lax.axis_index("s")
    wid = c * NS + s

    z = jnp.zeros((16,), jnp.float32)
    for j in range(CHUNK // 16):
        ones_v[pl.ds(j * 16, 16)] = jnp.ones((16,), jnp.float32)
    for j in range(ROWS_PER_TILE // 16):
        stage_v[pl.ds(j * 16, 16)] = z

    r0 = s * ROWS_PER_TILE
    pltpu.sync_copy(stage_v, acc_src.at[pl.ds(r0, ROWS_PER_TILE)])
    pltpu.sync_copy(stage_v, acc_dst.at[pl.ds(r0, ROWS_PER_TILE)])
    plsc.subcore_barrier()

    row_base = wid * DEG_ROWS_PER_TILE

    def body(b, carry):
        rb = row_base + b * DEG_BLOCK
        li = pltpu.async_copy(src_hbm.at[pl.ds(rb, DEG_BLOCK)], sbidx, semi)
        lj = pltpu.async_copy(dst_hbm.at[pl.ds(rb, DEG_BLOCK)], dbidx, semi)
        li.wait()
        lj.wait()
        for j in range(DEG_BLOCK):
            pltpu.async_copy(ones_v, acc_src.at[sbidx.at[j]], sems, add=True)
            pltpu.async_copy(ones_v, acc_dst.at[dbidx.at[j]], sems, add=True)
        for j in range(DEG_BLOCK):
            pltpu.make_async_copy(ones_v, acc_src.at[sbidx.at[j]], sems).wait()
            pltpu.make_async_copy(ones_v, acc_dst.at[dbidx.at[j]], sems).wait()
        return carry

    lax.fori_loop(0, DEG_NBLOCKS, body, 0)
    plsc.subcore_barrier()

    pltpu.sync_copy(acc_src.at[pl.ds(r0, ROWS_PER_TILE)],
                    out_hbm.at[c, 0, pl.ds(r0, ROWS_PER_TILE)])
    pltpu.sync_copy(acc_dst.at[pl.ds(r0, ROWS_PER_TILE)],
                    out_hbm.at[c, 1, pl.ds(r0, ROWS_PER_TILE)])


_degree_call = functools.partial(
    pl.kernel,
    mesh=_MESH,
    out_type=jax.ShapeDtypeStruct((NC, 2, N_PAD), jnp.float32),
    scratch_types=[
        pltpu.VMEM((DEG_BLOCK, CHUNK), jnp.int32),
        pltpu.VMEM((DEG_BLOCK, CHUNK), jnp.int32),
        pltpu.VMEM((CHUNK,), jnp.float32),
        pltpu.VMEM((ROWS_PER_TILE,), jnp.float32),
        pltpu.VMEM_SHARED((N_PAD,), jnp.float32),
        pltpu.VMEM_SHARED((N_PAD,), jnp.float32),
        pltpu.SemaphoreType.DMA,
        pltpu.SemaphoreType.DMA,
    ],
)(_degree_body)


def _agg_body(y_hbm, src_hbm, dst_hbm, out_hbm, sidx0, sidx1, didx0, didx1,
              rows0, rows1, acc, semi0, semi1, semg0, semg1):
    c = lax.axis_index("c")
    s = lax.axis_index("s")
    wid = c * NS + s

    z = jnp.zeros((16,), jnp.float32)

    def zbody(i, carry):
        for j in range(D // 16):
            rows0[i, pl.ds(j * 16, 16)] = z
        return carry

    lax.fori_loop(0, CHUNK, zbody, 0)
    r0 = s * ROWS_PER_TILE
    for j in range(ROWS_PER_TILE // CHUNK):
        pltpu.sync_copy(rows0, acc.at[pl.ds(r0 + j * CHUNK, CHUNK)])
    plsc.subcore_barrier()

    base = wid * E_PER_TILE

    def load_idx(off, sref, dref, sem):
        pltpu.async_copy(src_hbm.at[pl.ds(off, CHUNK)], sref, sem)
        pltpu.async_copy(dst_hbm.at[pl.ds(off, CHUNK)], dref, sem)

    def wait_idx(sref, dref, sem):
        pltpu.make_async_copy(src_hbm.at[pl.ds(0, CHUNK)], sref, sem).wait()
        pltpu.make_async_copy(dst_hbm.at[pl.ds(0, CHUNK)], dref, sem).wait()

    load_idx(base, sidx0, didx0, semi0)
    wait_idx(sidx0, didx0, semi0)
    pltpu.async_copy(y_hbm.at[sidx0], rows0, semg0)
    load_idx(base + CHUNK, sidx1, didx1, semi1)

    def pair(k, carry):
        a = 1 + 2 * k
        wait_idx(sidx1, didx1, semi1)
        pltpu.async_copy(y_hbm.at[sidx1], rows1, semg1)
        pltpu.make_async_copy(y_hbm.at[sidx0], rows0, semg0).wait()
        pltpu.sync_copy(rows0, acc.at[didx0], add=True)
        load_idx(base + (a + 1) * CHUNK, sidx0, didx0, semi0)
        wait_idx(sidx0, didx0, semi0)
        pltpu.async_copy(y_hbm.at[sidx0], rows0, semg0)
        pltpu.make_async_copy(y_hbm.at[sidx1], rows1, semg1).wait()
        pltpu.sync_copy(rows1, acc.at[didx1], add=True)
        load_idx(base + (a + 2) * CHUNK, sidx1, didx1, semi1)
        return carry

    lax.fori_loop(0, N_PAIRS, pair, 0)

    wait_idx(sidx1, didx1, semi1)
    pltpu.make_async_copy(y_hbm.at[sidx0], rows0, semg0).wait()
    pltpu.sync_copy(rows0, acc.at[didx0], add=True)
    plsc.subcore_barrier()

    pltpu.sync_copy(acc.at[pl.ds(r0, ROWS_PER_TILE)],
                    out_hbm.at[c, pl.ds(r0, ROWS_PER_TILE)])


_agg_call = functools.partial(
    pl.kernel,
    mesh=_MESH,
    out_type=jax.ShapeDtypeStruct((NC, N_PAD, D), jnp.float32),
    scratch_types=[
        pltpu.VMEM((CHUNK,), jnp.int32),
        pltpu.VMEM((CHUNK,), jnp.int32),
        pltpu.VMEM((CHUNK,), jnp.int32),
        pltpu.VMEM((CHUNK,), jnp.int32),
        pltpu.VMEM((CHUNK, D), jnp.float32),
        pltpu.VMEM((CHUNK, D), jnp.float32),
        pltpu.VMEM_SHARED((N_PAD, D), jnp.float32),
        pltpu.SemaphoreType.DMA,
        pltpu.SemaphoreType.DMA,
        pltpu.SemaphoreType.DMA,
        pltpu.SemaphoreType.DMA,
    ],
)(_agg_body)


def _t1_body(x_ref, w_ref, dc_ref, y_ref, so_ref, si_ref):
    cs = dc_ref[0, 0] + dc_ref[1, 0]
    cd = dc_ref[0, 1] + dc_ref[1, 1]
    so = lax.rsqrt(jnp.maximum(cs, 1.0))
    si = lax.rsqrt(jnp.maximum(cd, 1.0))
    y = jnp.dot(x_ref[...], w_ref[...], preferred_element_type=jnp.float32)
    y_ref[...] = y * so
    so_ref[...] = so
    si_ref[...] = si


def _t1_call(x, w, dcnt):
    return pl.pallas_call(
        _t1_body,
        out_shape=(
            jax.ShapeDtypeStruct((N_PAD, D), jnp.float32),
            jax.ShapeDtypeStruct((N_PAD, 1), jnp.float32),
            jax.ShapeDtypeStruct((N_PAD, 1), jnp.float32),
        ),
    )(x, w, dcnt)


def _tmid_body(p_ref, si_ref, so_ref, b_ref, w_ref, y_ref):
    h = (p_ref[0] + p_ref[1]) * si_ref[...] + b_ref[...]
    h = jnp.maximum(h, 0.0)
    y = jnp.dot(h, w_ref[...], preferred_element_type=jnp.float32)
    y_ref[...] = y * so_ref[...]


def _tmid_call(p, si, so, b, w):
    return pl.pallas_call(
        _tmid_body,
        out_shape=jax.ShapeDtypeStruct((N_PAD, D), jnp.float32),
    )(p, si, so, b, w)


def _t4_body(p_ref, si_ref, b_ref, o_ref):
    o_ref[...] = (p_ref[0] + p_ref[1]) * si_ref[...] + b_ref[...]


def _t4_call(p, si, b):
    return pl.pallas_call(
        _t4_body,
        out_shape=jax.ShapeDtypeStruct((N_PAD, D), jnp.float32),
    )(p, si, b)


def kernel(features, edge_index, W1, b1, W2, b2, W3, b3):
    src = edge_index[0].astype(jnp.int32)
    dst = edge_index[1].astype(jnp.int32)
    src_p = jnp.pad(src, (0, 2 * CHUNK))
    dst_p = jnp.pad(dst, (0, 2 * CHUNK))
    n_pad_rows = ROWS_TOTAL_PAD - ROWS_TOTAL
    pad_idx = (N_NODES + jnp.arange(n_pad_rows * CHUNK, dtype=jnp.int32)
               % (N_PAD - N_NODES)).reshape(n_pad_rows, CHUNK)
    src2d = jnp.concatenate([src.reshape(ROWS_TOTAL, CHUNK), pad_idx], 0)
    dst2d = jnp.concatenate([dst.reshape(ROWS_TOTAL, CHUNK), pad_idx], 0)
    x = jnp.pad(features, ((0, N_PAD - N_NODES), (0, 0)))

    dcnt = _degree_call(src2d, dst2d)
    dcnt = dcnt.reshape(NC, 2, N_PAD, 1)

    y1, so, si = _t1_call(x, W1, dcnt)
    p1 = _agg_call(y1, src_p, dst_p)
    y2 = _tmid_call(p1, si, so, b1.reshape(1, D), W2)
    p2 = _agg_call(y2, src_p, dst_p)
    y3 = _tmid_call(p2, si, so, b2.reshape(1, D), W3)
    p3 = _agg_call(y3, src_p, dst_p)
    out = _t4_call(p3, si, b3.reshape(1, D))
    return out[:N_NODES]


# speedup vs baseline: 12.2179x; 1.3297x over previous
import functools

import jax
import jax.numpy as jnp
from jax import lax
from jax.experimental import pallas as pl
from jax.experimental.pallas import tpu as pltpu
from jax.experimental.pallas import tpu_sc as plsc

N_NODES = 10000
N_EDGES = 320000
D = 128

NC = 2
NS = 16
NW = NC * NS

N_PAD = 10240
ROWS_PER_TILE = N_PAD // NS

E_PER_TILE = N_EDGES // NW
CHUNK = 80
N_CHUNKS = E_PER_TILE // CHUNK

DEG_BLOCK = 8
ROWS_TOTAL = N_EDGES // CHUNK
ROWS_TOTAL_PAD = 4096
DEG_ROWS_PER_TILE = ROWS_TOTAL_PAD // NW
DEG_NBLOCKS = DEG_ROWS_PER_TILE // DEG_BLOCK

_MESH = plsc.VectorSubcoreMesh(core_axis_name="c", subcore_axis_name="s")


def _degree_body(src_hbm, dst_hbm, out_hbm, sbidx, dbidx, ones_v, stage_v,
                 acc_src, acc_dst, semi, sems):
    c = lax.axis_index("c")
    s = lax.axis_index("s")
    wid = c * NS + s

    z = jnp.zeros((16,), jnp.float32)
    for j in range(CHUNK // 16):
        ones_v[pl.ds(j * 16, 16)] = jnp.ones((16,), jnp.float32)
    for j in range(ROWS_PER_TILE // 16):
        stage_v[pl.ds(j * 16, 16)] = z

    r0 = s * ROWS_PER_TILE
    pltpu.sync_copy(stage_v, acc_src.at[pl.ds(r0, ROWS_PER_TILE)])
    pltpu.sync_copy(stage_v, acc_dst.at[pl.ds(r0, ROWS_PER_TILE)])
    plsc.subcore_barrier()

    row_base = wid * DEG_ROWS_PER_TILE

    def body(b, carry):
        rb = row_base + b * DEG_BLOCK
        li = pltpu.async_copy(src_hbm.at[pl.ds(rb, DEG_BLOCK)], sbidx, semi)
        lj = pltpu.async_copy(dst_hbm.at[pl.ds(rb, DEG_BLOCK)], dbidx, semi)
        li.wait()
        lj.wait()
        for j in range(DEG_BLOCK):
            pltpu.async_copy(ones_v, acc_src.at[sbidx.at[j]], sems, add=True)
            pltpu.async_copy(ones_v, acc_dst.at[dbidx.at[j]], sems, add=True)
        for j in range(DEG_BLOCK):
            pltpu.make_async_copy(ones_v, acc_src.at[sbidx.at[j]], sems).wait()
            pltpu.make_async_copy(ones_v, acc_dst.at[dbidx.at[j]], sems).wait()
        return carry

    lax.fori_loop(0, DEG_NBLOCKS, body, 0)
    plsc.subcore_barrier()

    pltpu.sync_copy(acc_src.at[pl.ds(r0, ROWS_PER_TILE)],
                    out_hbm.at[c, 0, pl.ds(r0, ROWS_PER_TILE)])
    pltpu.sync_copy(acc_dst.at[pl.ds(r0, ROWS_PER_TILE)],
                    out_hbm.at[c, 1, pl.ds(r0, ROWS_PER_TILE)])


_degree_call = functools.partial(
    pl.kernel,
    mesh=_MESH,
    out_type=jax.ShapeDtypeStruct((NC, 2, N_PAD), jnp.float32),
    scratch_types=[
        pltpu.VMEM((DEG_BLOCK, CHUNK), jnp.int32),
        pltpu.VMEM((DEG_BLOCK, CHUNK), jnp.int32),
        pltpu.VMEM((CHUNK,), jnp.float32),
        pltpu.VMEM((ROWS_PER_TILE,), jnp.float32),
        pltpu.VMEM_SHARED((N_PAD,), jnp.float32),
        pltpu.VMEM_SHARED((N_PAD,), jnp.float32),
        pltpu.SemaphoreType.DMA,
        pltpu.SemaphoreType.DMA,
    ],
)(_degree_body)


E_PAD_TILE = 10080
N_CHUNKS_P = E_PAD_TILE // CHUNK
NBUF = 3
NIDX = 6
N_GROUPS = N_CHUNKS_P // NIDX


def _agg_body(y_hbm, src_hbm, dst3_hbm, out_hbm, didx,
              sb0, sb1, sb2, sb3, sb4, sb5, rb0, rb1, rb2, acc,
              i0, i1, i2, i3, i4, i5, g0, g1, g2, s0, s1, s2):
    c = lax.axis_index("c")
    s = lax.axis_index("s")
    wid = c * NS + s
    sbufs = [sb0, sb1, sb2, sb3, sb4, sb5]
    rbufs = [rb0, rb1, rb2]
    isems = [i0, i1, i2, i3, i4, i5]
    gsems = [g0, g1, g2]
    ssems = [s0, s1, s2]
    base = wid * E_PAD_TILE

    pltpu.async_copy(dst3_hbm.at[wid], didx, g0)

    z = jnp.zeros((16,), jnp.float32)

    def zbody(i, carry):
        for j in range(D // 16):
            rb0[i, pl.ds(j * 16, 16)] = z
        return carry

    lax.fori_loop(0, CHUNK, zbody, 0)
    r0 = s * ROWS_PER_TILE
    for j in range(ROWS_PER_TILE // CHUNK):
        pltpu.sync_copy(rb0, acc.at[pl.ds(r0 + j * CHUNK, CHUNK)])
    pltpu.make_async_copy(dst3_hbm.at[wid], didx, g0).wait()
    plsc.subcore_barrier()

    def sidx_load(i, b):
        pltpu.async_copy(src_hbm.at[pl.ds(base + i * CHUNK, CHUNK)],
                         sbufs[b], isems[b])

    def wait_sidx(i, b):
        pltpu.make_async_copy(src_hbm.at[pl.ds(base, CHUNK)],
                              sbufs[b], isems[b]).wait()

    def gather(b, rb):
        pltpu.async_copy(y_hbm.at[sbufs[b]], rbufs[rb], gsems[rb])

    def wait_gather(b, rb):
        pltpu.make_async_copy(y_hbm.at[sbufs[b]], rbufs[rb],
                              gsems[rb]).wait()

    def scatter(i, rb):
        pltpu.async_copy(rbufs[rb], acc.at[didx.at[i]], ssems[rb], add=True)

    def wait_scatter(i, rb):
        pltpu.make_async_copy(rbufs[rb], acc.at[didx.at[i]],
                              ssems[rb]).wait()

    for j in range(NIDX):
        sidx_load(j, j)
    wait_sidx(0, 0)
    gather(0, 0)
    wait_sidx(1, 1)
    gather(1, 1)

    def group(k, carry):
        for j in range(NIDX):
            i = k * NIDX + j
            rb = j % NBUF
            wait_gather(j, rb)

            @pl.when(i + NIDX < N_CHUNKS_P)
            def _():
                sidx_load(i + NIDX, j)

            scatter(i, rb)

            @pl.when(i >= 1)
            def _():
                wait_scatter(i - 1, (j - 1) % NBUF)

            @pl.when(i + 2 < N_CHUNKS_P)
            def _():
                wait_sidx(i + 2, (j + 2) % NIDX)
                gather((j + 2) % NIDX, (j + 2) % NBUF)
        return carry

    lax.fori_loop(0, N_GROUPS, group, 0)
    wait_scatter(N_CHUNKS_P - 1, (N_CHUNKS_P - 1) % NBUF)
    plsc.subcore_barrier()

    pltpu.sync_copy(acc.at[pl.ds(r0, ROWS_PER_TILE)],
                    out_hbm.at[c, pl.ds(r0, ROWS_PER_TILE)])


_agg_call = functools.partial(
    pl.kernel,
    mesh=_MESH,
    out_type=jax.ShapeDtypeStruct((NC, N_PAD, D), jnp.float32),
    scratch_types=(
        [pltpu.VMEM((N_CHUNKS_P, CHUNK), jnp.int32)]
        + [pltpu.VMEM((CHUNK,), jnp.int32) for _ in range(NIDX)]
        + [pltpu.VMEM((CHUNK, D), jnp.float32) for _ in range(NBUF)]
        + [pltpu.VMEM_SHARED((N_PAD, D), jnp.float32)]
        + [pltpu.SemaphoreType.DMA for _ in range(NIDX + 2 * NBUF)]
    ),
)(_agg_body)


def _t1_body(x_ref, w_ref, dc_ref, y_ref, so_ref, si_ref):
    cs = dc_ref[0, 0] + dc_ref[1, 0]
    cd = dc_ref[0, 1] + dc_ref[1, 1]
    so = lax.rsqrt(jnp.maximum(cs, 1.0))
    si = lax.rsqrt(jnp.maximum(cd, 1.0))
    y = jnp.dot(x_ref[...], w_ref[...], preferred_element_type=jnp.float32)
    y_ref[...] = y * so
    so_ref[...] = so
    si_ref[...] = si


def _t1_call(x, w, dcnt):
    return pl.pallas_call(
        _t1_body,
        out_shape=(
            jax.ShapeDtypeStruct((N_PAD, D), jnp.float32),
            jax.ShapeDtypeStruct((N_PAD, 1), jnp.float32),
            jax.ShapeDtypeStruct((N_PAD, 1), jnp.float32),
        ),
    )(x, w, dcnt)


def _tmid_body(p_ref, si_ref, so_ref, b_ref, w_ref, y_ref):
    h = (p_ref[0] + p_ref[1]) * si_ref[...] + b_ref[...]
    h = jnp.maximum(h, 0.0)
    y = jnp.dot(h, w_ref[...], preferred_element_type=jnp.float32)
    y_ref[...] = y * so_ref[...]


def _tmid_call(p, si, so, b, w):
    return pl.pallas_call(
        _tmid_body,
        out_shape=jax.ShapeDtypeStruct((N_PAD, D), jnp.float32),
    )(p, si, so, b, w)


def _t4_body(p_ref, si_ref, b_ref, o_ref):
    o_ref[...] = (p_ref[0] + p_ref[1]) * si_ref[...] + b_ref[...]


def _t4_call(p, si, b):
    return pl.pallas_call(
        _t4_body,
        out_shape=jax.ShapeDtypeStruct((N_PAD, D), jnp.float32),
    )(p, si, b)


def kernel(features, edge_index, W1, b1, W2, b2, W3, b3):
    src = edge_index[0].astype(jnp.int32)
    dst = edge_index[1].astype(jnp.int32)
    n_pad_e = E_PAD_TILE - E_PER_TILE
    epad = (N_NODES + jnp.arange(NW * n_pad_e, dtype=jnp.int32)
            % (N_PAD - N_NODES)).reshape(NW, n_pad_e)
    src_w = jnp.concatenate([src.reshape(NW, E_PER_TILE), epad], axis=1)
    dst_w = jnp.concatenate([dst.reshape(NW, E_PER_TILE), epad], axis=1)
    src_flat = src_w.reshape(-1)
    dst3 = dst_w.reshape(NW, N_CHUNKS_P, CHUNK)
    n_pad_rows = ROWS_TOTAL_PAD - ROWS_TOTAL
    pad_idx = (N_NODES + jnp.arange(n_pad_rows * CHUNK, dtype=jnp.int32)
               % (N_PAD - N_NODES)).reshape(n_pad_rows, CHUNK)
    src2d = jnp.concatenate([src.reshape(ROWS_TOTAL, CHUNK), pad_idx], 0)
    dst2d = jnp.concatenate([dst.reshape(ROWS_TOTAL, CHUNK), pad_idx], 0)
    x = jnp.pad(features, ((0, N_PAD - N_NODES), (0, 0)))

    dcnt = _degree_call(src2d, dst2d)
    dcnt = dcnt.reshape(NC, 2, N_PAD, 1)

    y1, so, si = _t1_call(x, W1, dcnt)
    p1 = _agg_call(y1, src_flat, dst3)
    y2 = _tmid_call(p1, si, so, b1.reshape(1, D), W2)
    p2 = _agg_call(y2, src_flat, dst3)
    y3 = _tmid_call(p2, si, so, b2.reshape(1, D), W3)
    p3 = _agg_call(y3, src_flat, dst3)
    out = _t4_call(p3, si, b3.reshape(1, D))
    return out[:N_NODES]
